# baseline (speedup 1.0000x reference)
import functools

import jax
import jax.numpy as jnp
import numpy as np
from jax import lax
from jax.experimental import pallas as pl
from jax.experimental.pallas import tpu as pltpu
from jax.experimental.pallas import tpu_sc as plsc

B, N, T, H = 16, 256, 128, 16
LANES = 16
H_PER = 4
NHG = H // H_PER
B_GRP = B // 8
CHUNK = 16
NCHUNK = N // CHUNK
NITEMS = B_GRP * NCHUNK
NJB = N // LANES

_FMAX = np.float32(np.finfo(np.float32).max)
_FMIN = np.float32(np.finfo(np.float32).min)

_TAKE_DNUMS = lax.GatherDimensionNumbers(
    offset_dims=(), collapsed_slice_dims=(0,), start_index_map=(0,)
)


def _lane_splat(vec, lane):
    idx = jnp.broadcast_to(lane, (LANES,)).astype(jnp.int32)
    return lax.gather(
        vec,
        idx[:, None],
        dimension_numbers=_TAKE_DNUMS,
        slice_sizes=(1,),
        mode=lax.GatherScatterMode.PROMISE_IN_BOUNDS,
    )


def _logd_pass(d):
    def body(d_ref, o_ref):
        x = jnp.log(d_ref[...])
        x = jnp.where(jnp.isnan(x), _FMAX, x)
        o_ref[...] = jnp.clip(x, _FMIN, _FMAX)

    return pl.pallas_call(
        body,
        grid=(d.shape[0],),
        in_specs=[pl.BlockSpec((1, N, N), lambda b: (b, 0, 0))],
        out_specs=pl.BlockSpec((1, N, N), lambda b: (b, 0, 0)),
        out_shape=jax.ShapeDtypeStruct(d.shape, jnp.float32),
    )(d)


_mesh = plsc.VectorSubcoreMesh(core_axis_name="c", subcore_axis_name="s")


@functools.partial(
    pl.kernel,
    mesh=_mesh,
    out_type=jax.ShapeDtypeStruct((B, H, N, N), jnp.float32),
    scratch_types=[
        (pltpu.VMEM((T * T,), jnp.float32),) * H_PER,
        pltpu.VMEM((B_GRP, N), jnp.int32),
        pltpu.VMEM((CHUNK, N), jnp.float32),
        pltpu.VMEM((CHUNK, N), jnp.float32),
        pltpu.VMEM((H_PER, CHUNK, N), jnp.float32),
        pltpu.VMEM((H_PER, CHUNK, N), jnp.float32),
        pltpu.SemaphoreType.DMA,
        pltpu.SemaphoreType.DMA,
        pltpu.SemaphoreType.DMA,
        pltpu.SemaphoreType.DMA,
    ],
    compiler_params=pltpu.CompilerParams(needs_layout_passes=False),
)
def _sc_pass(logd_hbm, tok_hbm, pt_hbm, out_hbm,
             p_h, tok_v, ld0, ld1, ob0, ob1, is0, is1, os0, os1):
    c = lax.axis_index("c")
    s = lax.axis_index("s")
    hg = s % NHG
    h0 = hg * H_PER
    bg = c * 4 + s // NHG
    ld = (ld0, ld1)
    ob = (ob0, ob1)
    isem = (is0, is1)
    osem = (os0, os1)

    for hh in range(H_PER):
        pltpu.sync_copy(pt_hbm.at[h0 + hh], p_h[hh])
    pltpu.sync_copy(tok_hbm.at[pl.ds(bg * B_GRP, B_GRP)], tok_v)

    def item_bcc(g):
        lb = g // NCHUNK
        return lb, bg * B_GRP + lb, g % NCHUNK

    def start_in(g, par):
        _, b, cc = item_bcc(g)
        pltpu.make_async_copy(
            logd_hbm.at[b, pl.ds(cc * CHUNK, CHUNK)], ld[par], isem[par]
        ).start()

    start_in(0, 0)

    def pair_body(k, carry):
        for par in (0, 1):
            g = k * 2 + par
            lb, b, cc = item_bcc(g)

            @pl.when(g + 1 < NITEMS)
            def _():
                start_in(g + 1, 1 - par)

            pltpu.make_async_copy(
                logd_hbm.at[b, pl.ds(cc * CHUNK, CHUNK)], ld[par], isem[par]
            ).wait()

            @pl.when(g >= 2)
            def _():
                pltpu.make_async_copy(
                    ob[par],
                    out_hbm.at[b, pl.ds(h0, H_PER), pl.ds(cc * CHUNK, CHUNK)],
                    osem[par],
                ).wait()

            tjs = [tok_v[lb, pl.ds(jb * LANES, LANES)] for jb in range(NJB)]
            ldb = ld[par]
            obb = ob[par]
            tiv = tok_v[lb, pl.ds(cc * CHUNK, LANES)]

            @plsc.parallel_loop(0, CHUNK)
            def i_loop(il, tiv=tiv, ldb=ldb, obb=obb, tjs=tjs):
                base = _lane_splat(tiv, il) * T
                for jb in range(NJB):
                    idx = base + tjs[jb]
                    lvec = ldb[il, pl.ds(jb * LANES, LANES)]
                    for hh in range(H_PER):
                        gv = plsc.load_gather(p_h[hh], [idx])
                        obb[hh, il, pl.ds(jb * LANES, LANES)] = gv * lvec

            pltpu.make_async_copy(
                obb,
                out_hbm.at[b, pl.ds(h0, H_PER), pl.ds(cc * CHUNK, CHUNK)],
                osem[par],
            ).start()
        return carry

    lax.fori_loop(0, NITEMS // 2, pair_body, 0)

    for par in (0, 1):
        g = NITEMS - 2 + par
        _, b, cc = item_bcc(g)
        pltpu.make_async_copy(
            ob[par],
            out_hbm.at[b, pl.ds(h0, H_PER), pl.ds(cc * CHUNK, CHUNK)],
            osem[par],
        ).wait()


def kernel(d, tokens, p_table):
    logd = _logd_pass(d)
    pt = jnp.transpose(p_table, (2, 0, 1)).reshape(H, T * T)
    tok = tokens.astype(jnp.int32)
    return _sc_pass(logd, tok, pt)


# speedup vs baseline: 1.1914x; 1.1914x over previous
import functools

import jax
import jax.numpy as jnp
import numpy as np
from jax import lax
from jax.experimental import pallas as pl
from jax.experimental.pallas import tpu as pltpu
from jax.experimental.pallas import tpu_sc as plsc

B, N, T, H = 16, 256, 128, 16
LANES = 16
H_PER = 2
CHUNK = 32
NCHUNK = N // CHUNK
B_QUAD = B // 4
NITEMS = B_QUAD * NCHUNK
NJB = N // LANES
NQ = CHUNK // LANES

_FMAX = np.float32(np.finfo(np.float32).max)
_FMIN = np.float32(np.finfo(np.float32).min)

_TAKE_DNUMS = lax.GatherDimensionNumbers(
    offset_dims=(), collapsed_slice_dims=(0,), start_index_map=(0,)
)


def _lane_splat(vec, lane):
    idx = jnp.broadcast_to(lane, (LANES,)).astype(jnp.int32)
    return lax.gather(
        vec,
        idx[:, None],
        dimension_numbers=_TAKE_DNUMS,
        slice_sizes=(1,),
        mode=lax.GatherScatterMode.PROMISE_IN_BOUNDS,
    )


def _logd_pass(d):
    def body(d_ref, o_ref):
        x = jnp.log(d_ref[...])
        x = jnp.where(jnp.isnan(x), _FMAX, x)
        o_ref[...] = jnp.clip(x, _FMIN, _FMAX)

    return pl.pallas_call(
        body,
        grid=(d.shape[0],),
        in_specs=[pl.BlockSpec((1, N, N), lambda b: (b, 0, 0))],
        out_specs=pl.BlockSpec((1, N, N), lambda b: (b, 0, 0)),
        out_shape=jax.ShapeDtypeStruct(d.shape, jnp.float32),
    )(d)


_mesh = plsc.VectorSubcoreMesh(core_axis_name="c", subcore_axis_name="s")


@functools.partial(
    pl.kernel,
    mesh=_mesh,
    out_type=jax.ShapeDtypeStruct((B, H, N, N), jnp.float32),
    scratch_types=[
        pltpu.VMEM((T * T,), jnp.float32),
        pltpu.VMEM((T * T,), jnp.float32),
        pltpu.VMEM((B_QUAD, N), jnp.int32),
        pltpu.VMEM((CHUNK, N), jnp.float32),
        pltpu.VMEM((CHUNK, N), jnp.float32),
        pltpu.VMEM((H_PER, CHUNK, N), jnp.float32),
        pltpu.VMEM((H_PER, CHUNK, N), jnp.float32),
        pltpu.SemaphoreType.DMA,
        pltpu.SemaphoreType.DMA,
        pltpu.SemaphoreType.DMA,
        pltpu.SemaphoreType.DMA,
    ],
    compiler_params=pltpu.CompilerParams(needs_layout_passes=False),
)
def _sc_pass(logd_hbm, tok_hbm, pt_hbm, out_hbm,
             p_h0, p_h1, tok_v, ld0, ld1, ob0, ob1, is0, is1, os0, os1):
    c = lax.axis_index("c")
    s = lax.axis_index("s")
    h0 = (s % 8) * H_PER
    bq = c * 2 + s // 8
    ld = (ld0, ld1)
    ob = (ob0, ob1)
    isem = (is0, is1)
    osem = (os0, os1)

    pltpu.sync_copy(pt_hbm.at[h0], p_h0)
    pltpu.sync_copy(pt_hbm.at[h0 + 1], p_h1)
    pltpu.sync_copy(tok_hbm.at[pl.ds(bq * B_QUAD, B_QUAD)], tok_v)

    def item_bcc(g):
        lb = g // NCHUNK
        return lb, bq * B_QUAD + lb, g % NCHUNK

    def start_in(g, par):
        _, b, cc = item_bcc(g)
        pltpu.make_async_copy(
            logd_hbm.at[b, pl.ds(cc * CHUNK, CHUNK)], ld[par], isem[par]
        ).start()

    start_in(0, 0)

    def pair_body(k, carry):
        for par in (0, 1):
            g = k * 2 + par
            lb, b, cc = item_bcc(g)

            @pl.when(g + 1 < NITEMS)
            def _():
                start_in(g + 1, 1 - par)

            pltpu.make_async_copy(
                logd_hbm.at[b, pl.ds(cc * CHUNK, CHUNK)], ld[par], isem[par]
            ).wait()

            @pl.when(g >= 2)
            def _():
                pltpu.make_async_copy(
                    ob[par],
                    out_hbm.at[b, pl.ds(h0, H_PER), pl.ds(cc * CHUNK, CHUNK)],
                    osem[par],
                ).wait()

            tjs = [tok_v[lb, pl.ds(jb * LANES, LANES)] for jb in range(NJB)]
            ldb = ld[par]
            obb = ob[par]

            for q in range(NQ):
                tiv = tok_v[lb, pl.ds(cc * CHUNK + q * LANES, LANES)]

                @plsc.parallel_loop(0, LANES)
                def i_loop(r, tiv=tiv, q=q, ldb=ldb, obb=obb, tjs=tjs):
                    il = q * LANES + r
                    base = _lane_splat(tiv, r) * T
                    for jb in range(NJB):
                        idx = base + tjs[jb]
                        lvec = ldb[il, pl.ds(jb * LANES, LANES)]
                        g0 = plsc.load_gather(p_h0, [idx])
                        g1 = plsc.load_gather(p_h1, [idx])
                        obb[0, il, pl.ds(jb * LANES, LANES)] = g0 * lvec
                        obb[1, il, pl.ds(jb * LANES, LANES)] = g1 * lvec

            pltpu.make_async_copy(
                obb,
                out_hbm.at[b, pl.ds(h0, H_PER), pl.ds(cc * CHUNK, CHUNK)],
                osem[par],
            ).start()
        return carry

    lax.fori_loop(0, NITEMS // 2, pair_body, 0)

    for par in (0, 1):
        g = NITEMS - 2 + par
        _, b, cc = item_bcc(g)
        pltpu.make_async_copy(
            ob[par],
            out_hbm.at[b, pl.ds(h0, H_PER), pl.ds(cc * CHUNK, CHUNK)],
            osem[par],
        ).wait()


def kernel(d, tokens, p_table):
    logd = _logd_pass(d)
    pt = jnp.transpose(p_table, (2, 0, 1)).reshape(H, T * T)
    tok = tokens.astype(jnp.int32)
    return _sc_pass(logd, tok, pt)


# speedup vs baseline: 1.2585x; 1.0563x over previous
import functools

import jax
import jax.numpy as jnp
import numpy as np
from jax import lax
from jax.experimental import pallas as pl
from jax.experimental.pallas import tpu as pltpu
from jax.experimental.pallas import tpu_sc as plsc

B, N, T, H = 16, 256, 128, 16
LANES = 16
H_PER = 4
CHUNK = 16
NCHUNK = N // CHUNK
B_GRP = B // 8
NITEMS = B_GRP * NCHUNK
NJB = N // LANES

_FMAX = np.float32(np.finfo(np.float32).max)
_FMIN = np.float32(np.finfo(np.float32).min)

_TAKE_DNUMS = lax.GatherDimensionNumbers(
    offset_dims=(), collapsed_slice_dims=(0,), start_index_map=(0,)
)


def _lane_splat(vec, lane):
    idx = jnp.broadcast_to(lane, (LANES,)).astype(jnp.int32)
    return lax.gather(
        vec,
        idx[:, None],
        dimension_numbers=_TAKE_DNUMS,
        slice_sizes=(1,),
        mode=lax.GatherScatterMode.PROMISE_IN_BOUNDS,
    )


def _logd_pass(d):
    def body(d_ref, o_ref):
        x = jnp.log(d_ref[...])
        x = jnp.where(jnp.isnan(x), _FMAX, x)
        o_ref[...] = jnp.clip(x, _FMIN, _FMAX)

    return pl.pallas_call(
        body,
        grid=(d.shape[0],),
        in_specs=[pl.BlockSpec((1, N, N), lambda b: (b, 0, 0))],
        out_specs=pl.BlockSpec((1, N, N), lambda b: (b, 0, 0)),
        out_shape=jax.ShapeDtypeStruct(d.shape, jnp.float32),
    )(d)


_mesh = plsc.VectorSubcoreMesh(core_axis_name="c", subcore_axis_name="s")


@functools.partial(
    pl.kernel,
    mesh=_mesh,
    out_type=jax.ShapeDtypeStruct((B, H, N, N), jnp.float32),
    scratch_types=[
        pltpu.VMEM((T * T,), jnp.float32),
        pltpu.VMEM((T * T,), jnp.float32),
        pltpu.VMEM((T * T,), jnp.float32),
        pltpu.VMEM((T * T,), jnp.float32),
        pltpu.VMEM((B_GRP, N), jnp.int32),
        pltpu.VMEM((CHUNK, N), jnp.float32),
        pltpu.VMEM((CHUNK, N), jnp.float32),
        pltpu.VMEM((H_PER, CHUNK, N), jnp.float32),
        pltpu.VMEM((H_PER, CHUNK, N), jnp.float32),
        pltpu.SemaphoreType.DMA,
        pltpu.SemaphoreType.DMA,
        pltpu.SemaphoreType.DMA,
        pltpu.SemaphoreType.DMA,
    ],
    compiler_params=pltpu.CompilerParams(needs_layout_passes=False),
)
def _sc_pass(logd_hbm, tok_hbm, pt_hbm, out_hbm,
             p_h0, p_h1, p_h2, p_h3, tok_v, ld0, ld1, ob0, ob1,
             is0, is1, os0, os1):
    c = lax.axis_index("c")
    s = lax.axis_index("s")
    h0 = (s % 4) * H_PER
    bg = c * 4 + s // 4
    ld = (ld0, ld1)
    ob = (ob0, ob1)
    isem = (is0, is1)
    osem = (os0, os1)
    planes = (p_h0, p_h1, p_h2, p_h3)

    for k in range(H_PER):
        pltpu.sync_copy(pt_hbm.at[h0 + k], planes[k])
    pltpu.sync_copy(tok_hbm.at[pl.ds(bg * B_GRP, B_GRP)], tok_v)

    def item_bcc(g):
        lb = g // NCHUNK
        return lb, bg * B_GRP + lb, g % NCHUNK

    def start_in(g, par):
        _, b, cc = item_bcc(g)
        pltpu.make_async_copy(
            logd_hbm.at[b, pl.ds(cc * CHUNK, CHUNK)], ld[par], isem[par]
        ).start()

    start_in(0, 0)

    def pair_body(k, carry):
        for par in (0, 1):
            g = k * 2 + par
            lb, b, cc = item_bcc(g)

            @pl.when(g + 1 < NITEMS)
            def _():
                start_in(g + 1, 1 - par)

            pltpu.make_async_copy(
                logd_hbm.at[b, pl.ds(cc * CHUNK, CHUNK)], ld[par], isem[par]
            ).wait()

            @pl.when(g >= 2)
            def _():
                pltpu.make_async_copy(
                    ob[par],
                    out_hbm.at[b, pl.ds(h0, H_PER), pl.ds(cc * CHUNK, CHUNK)],
                    osem[par],
                ).wait()

            tjs = [tok_v[lb, pl.ds(jb * LANES, LANES)] for jb in range(NJB)]
            ldb = ld[par]
            obb = ob[par]
            tiv = tok_v[lb, pl.ds(cc * CHUNK, CHUNK)]

            @plsc.parallel_loop(0, CHUNK)
            def i_loop(r, tiv=tiv, ldb=ldb, obb=obb, tjs=tjs):
                base = _lane_splat(tiv, r) * T
                for jb in range(NJB):
                    idx = base + tjs[jb]
                    lvec = ldb[r, pl.ds(jb * LANES, LANES)]
                    g0 = plsc.load_gather(p_h0, [idx])
                    g1 = plsc.load_gather(p_h1, [idx])
                    g2 = plsc.load_gather(p_h2, [idx])
                    g3 = plsc.load_gather(p_h3, [idx])
                    obb[0, r, pl.ds(jb * LANES, LANES)] = g0 * lvec
                    obb[1, r, pl.ds(jb * LANES, LANES)] = g1 * lvec
                    obb[2, r, pl.ds(jb * LANES, LANES)] = g2 * lvec
                    obb[3, r, pl.ds(jb * LANES, LANES)] = g3 * lvec

            pltpu.make_async_copy(
                obb,
                out_hbm.at[b, pl.ds(h0, H_PER), pl.ds(cc * CHUNK, CHUNK)],
                osem[par],
            ).start()
        return carry

    lax.fori_loop(0, NITEMS // 2, pair_body, 0)

    for par in (0, 1):
        g = NITEMS - 2 + par
        _, b, cc = item_bcc(g)
        pltpu.make_async_copy(
            ob[par],
            out_hbm.at[b, pl.ds(h0, H_PER), pl.ds(cc * CHUNK, CHUNK)],
            osem[par],
        ).wait()


def kernel(d, tokens, p_table):
    logd = _logd_pass(d)
    pt = jnp.transpose(p_table, (2, 0, 1)).reshape(H, T * T)
    tok = tokens.astype(jnp.int32)
    return _sc_pass(logd, tok, pt)
